# baseline (speedup 1.0000x reference)
import jax
import jax.numpy as jnp
from jax.experimental import pallas as pl


def _copy_body(enc_ref, out_ref):
    out_ref[...] = enc_ref[...]


def kernel(x, encoding):
    batch_size, seq_length = x.shape
    d_model = encoding.shape[1]
    block_rows = 512
    grid = (seq_length // block_rows,)
    return pl.pallas_call(
        _copy_body,
        grid=grid,
        in_specs=[pl.BlockSpec((block_rows, d_model), lambda i: (i, 0))],
        out_specs=pl.BlockSpec((block_rows, d_model), lambda i: (i, 0)),
        out_shape=jax.ShapeDtypeStruct((seq_length, d_model), encoding.dtype),
    )(encoding)
